# Initial kernel scaffold; baseline (speedup 1.0000x reference)
import jax
import jax.numpy as jnp
from jax.experimental import pallas as pl


def kernel(x, batch, layer_idx, embedding, W1, b1, W2, b2):
    raise NotImplementedError("write your pallas kernel here")


# speedup vs baseline: 1.1185x; 1.1185x over previous
import functools

import jax
import jax.numpy as jnp
from jax import lax
from jax.experimental import pallas as pl
from jax.experimental.pallas import tpu as pltpu
from jax.experimental.pallas import tpu_sc as plsc

N = 50000
H = 256
G = 128
ROWS = 80
NBLOCKS = N // ROWS
NW = 32
LANES = 16
CHUNKS = H // LANES

_mesh = plsc.VectorSubcoreMesh(core_axis_name="c", subcore_axis_name="s")


def _num_blocks(wid):
    return jnp.where(wid < NBLOCKS - (NBLOCKS // NW) * NW, NBLOCKS // NW + 1,
                     NBLOCKS // NW).astype(jnp.int32)


@functools.partial(
    pl.kernel,
    out_type=jax.ShapeDtypeStruct((2, G, H), jnp.float32),
    mesh=_mesh,
    scratch_types=[
        pltpu.VMEM((ROWS, H), jnp.float32),
        pltpu.VMEM((ROWS,), jnp.int32),
        pltpu.VMEM((G, H), jnp.float32),
        pltpu.VMEM((G // 16, H), jnp.float32),
        pltpu.VMEM((G // 16, H), jnp.float32),
        pltpu.VMEM_SHARED((16, G, H), jnp.float32),
    ],
)
def _pool(x_hbm, batch_hbm, out_hbm, rowbuf, idxbuf, acc, stage, tmp, slots_sh):
    cid = lax.axis_index("c")
    sid = lax.axis_index("s")
    wid = sid * 2 + cid

    zero = jnp.zeros((LANES,), jnp.float32)

    def zbody(g, carry):
        for j in range(CHUNKS):
            acc[g, pl.ds(j * LANES, LANES)] = zero
        return carry

    lax.fori_loop(0, G, zbody, 0)

    def body(i, carry):
        base = (wid + i * NW) * ROWS
        pltpu.sync_copy(batch_hbm.at[pl.ds(base, ROWS)], idxbuf)
        pltpu.sync_copy(x_hbm.at[pl.ds(base, ROWS), :], rowbuf)
        for k in range(ROWS // 16):
            segs = idxbuf[pl.ds(k * 16, 16)]
            for r in range(16):
                seg = segs[r]
                row = k * 16 + r
                for j in range(CHUNKS):
                    sl = pl.ds(j * LANES, LANES)
                    plsc.addupdate(acc.at[seg, sl], rowbuf[row, sl])
        return carry

    lax.fori_loop(0, _num_blocks(wid), body, 0)

    pltpu.sync_copy(acc, slots_sh.at[sid])
    plsc.subcore_barrier()
    rbase = sid * (G // 16)
    pltpu.sync_copy(slots_sh.at[0, pl.ds(rbase, G // 16), :], stage)

    def rbody(t, carry):
        pltpu.sync_copy(slots_sh.at[t, pl.ds(rbase, G // 16), :], tmp)
        def gbody(g, c2):
            for j in range(CHUNKS):
                sl = pl.ds(j * LANES, LANES)
                plsc.addupdate(stage.at[g, sl], tmp[g, sl])
            return c2
        lax.fori_loop(0, G // 16, gbody, 0)
        return carry

    lax.fori_loop(1, 16, rbody, 0)
    pltpu.sync_copy(stage, out_hbm.at[cid, pl.ds(rbase, G // 16), :])


def _mlp_body(pp_ref, w1_ref, b1_ref, w2_ref, b2_ref, out_ref):
    vnb = pp_ref[0] + pp_ref[1]
    h = lax.dot_general(vnb, w1_ref[...], (((1,), (1,)), ((), ())),
                        preferred_element_type=jnp.float32) + b1_ref[...]
    h = 0.5 * h * (1.0 + lax.erf(h * jnp.float32(0.7071067811865476)))
    out_ref[...] = lax.dot_general(h, w2_ref[...], (((1,), (1,)), ((), ())),
                                   preferred_element_type=jnp.float32) + b2_ref[...]


@functools.partial(
    pl.kernel,
    out_type=jax.ShapeDtypeStruct((N, H), jnp.float32),
    mesh=_mesh,
    scratch_types=[
        pltpu.VMEM((ROWS, H), jnp.float32),
        pltpu.VMEM((ROWS, H), jnp.float32),
        pltpu.VMEM((ROWS,), jnp.int32),
        pltpu.SemaphoreType.DMA,
    ],
)
def _redistribute(x_hbm, batch_hbm, vn_hbm, out_hbm, rowbuf, vnrows, idxbuf, sem):
    cid = lax.axis_index("c")
    sid = lax.axis_index("s")
    wid = sid * 2 + cid

    def body(i, carry):
        b = wid + i * NW
        base = b * ROWS
        pltpu.sync_copy(batch_hbm.at[pl.ds(base, ROWS)], idxbuf)
        gather = pltpu.async_copy(vn_hbm.at[idxbuf], vnrows, sem)
        pltpu.sync_copy(x_hbm.at[pl.ds(base, ROWS), :], rowbuf)
        gather.wait()

        def row_body(r, c):
            for j in range(CHUNKS):
                sl = pl.ds(j * LANES, LANES)
                plsc.addupdate(rowbuf.at[r, sl], vnrows[r, sl])
            return c

        lax.fori_loop(0, ROWS, row_body, 0)
        pltpu.sync_copy(rowbuf, out_hbm.at[pl.ds(base, ROWS), :])
        return carry

    lax.fori_loop(0, _num_blocks(wid), body, 0)


def kernel(x, batch, layer_idx, embedding, W1, b1, W2, b2):
    batch32 = batch.astype(jnp.int32)
    partials = _pool(x, batch32)
    vn_updated = pl.pallas_call(
        _mlp_body,
        out_shape=jax.ShapeDtypeStruct((G, H), jnp.float32),
    )(partials, W1, b1.reshape(1, H), W2, b2.reshape(1, H))
    x_out = _redistribute(x, batch32, vn_updated)
    return (x_out, vn_updated)
